# baseline (speedup 1.0000x reference)
import functools

import jax
import jax.numpy as jnp
from jax import lax
from jax.experimental import pallas as pl
from jax.experimental.pallas import tpu as pltpu
from jax.experimental.pallas import tpu_sc as plsc

BATCH = 16384
HIST = 50
DIM = 32
NC = 2
NS = 16
NW = NC * NS

SEQ_TOTAL = BATCH * HIST
SEQ_PER_W = SEQ_TOTAL // NW
SEQ_ROWS = SEQ_PER_W // 128
INNER = 20
CHUNK = INNER * 128
OUTER = SEQ_PER_W // CHUNK
B_PER_W = BATCH // NW
B_ROWS = B_PER_W // 128


def _sc_gather(seq_idx, uidx, iidx, src_uid, src_iid, tgt_iid):
    mesh = plsc.VectorSubcoreMesh(core_axis_name="c", subcore_axis_name="s",
                                  num_cores=NC, num_subcores=NS)

    def body(seq_hbm, uidx_hbm, iidx_hbm, uid_tab, siid_tab, tiid_tab,
             ufea_out, u_out, iid_out,
             idx_v, rows_v, uidx_v, iidx_v, sem):
        wid = lax.axis_index("s") * NC + lax.axis_index("c")

        pltpu.sync_copy(seq_hbm.at[wid], idx_v)
        base = wid * SEQ_PER_W

        def group(g, carry):
            cps = []
            for j in range(INNER):
                cps.append(pltpu.async_copy(
                    siid_tab.at[idx_v.at[g * INNER + j]],
                    rows_v.at[pl.ds(j * 128, 128)], sem))
            for cp in cps:
                cp.wait()
            pltpu.sync_copy(rows_v, ufea_out.at[pl.ds(base + g * CHUNK, CHUNK)])
            return carry

        lax.fori_loop(0, OUTER, group, 0, unroll=False)

        bbase = wid * B_PER_W
        pltpu.sync_copy(uidx_hbm.at[wid], uidx_v)
        cps = [pltpu.async_copy(uid_tab.at[uidx_v.at[j]],
                                rows_v.at[pl.ds(j * 128, 128)], sem)
               for j in range(B_ROWS)]
        for cp in cps:
            cp.wait()
        pltpu.sync_copy(rows_v.at[pl.ds(0, B_PER_W)],
                        u_out.at[pl.ds(bbase, B_PER_W)])

        pltpu.sync_copy(iidx_hbm.at[wid], iidx_v)
        cps = [pltpu.async_copy(tiid_tab.at[iidx_v.at[j]],
                                rows_v.at[pl.ds(j * 128, 128)], sem)
               for j in range(B_ROWS)]
        for cp in cps:
            cp.wait()
        pltpu.sync_copy(rows_v.at[pl.ds(0, B_PER_W)],
                        iid_out.at[pl.ds(bbase, B_PER_W)])

    run = pl.kernel(
        body,
        out_type=(
            jax.ShapeDtypeStruct((SEQ_TOTAL, DIM), jnp.float32),
            jax.ShapeDtypeStruct((BATCH, DIM), jnp.float32),
            jax.ShapeDtypeStruct((BATCH, DIM), jnp.float32),
        ),
        mesh=mesh,
        scratch_types=[
            pltpu.VMEM((SEQ_ROWS, 128), jnp.int32),
            pltpu.VMEM((CHUNK, DIM), jnp.float32),
            pltpu.VMEM((B_ROWS, 128), jnp.int32),
            pltpu.VMEM((B_ROWS, 128), jnp.int32),
            pltpu.SemaphoreType.DMA,
        ],
        compiler_params=pltpu.CompilerParams(use_tc_tiling_on_sc=False),
    )
    return run(seq_idx, uidx, iidx, src_uid, src_iid, tgt_iid)


def _tc_body(ufea_ref, seq_ref, u_ref, iid_ref,
             w1_ref, b1_ref, w2_ref, wd1_ref, bd1_ref, wd2_ref, bd2_ref,
             out_ref, *, bb):
    uf3 = ufea_ref[...]
    uf = uf3.reshape(bb * HIST, DIM)
    h = jnp.maximum(
        jnp.dot(uf, w1_ref[...], preferred_element_type=jnp.float32)
        + b1_ref[...], 0.0)
    e = jnp.sum(h * w2_ref[...], axis=1).reshape(bb, HIST)
    t = e - (seq_ref[...] == 0).astype(jnp.float32) * 1e8
    m = jnp.max(t, axis=1, keepdims=True)
    p = jnp.exp(t - m)
    att = p / jnp.sum(p, axis=1, keepdims=True)
    his = jnp.sum(uf3 * att[:, :, None], axis=1)
    g = jnp.maximum(
        jnp.dot(his, wd1_ref[...], preferred_element_type=jnp.float32)
        + bd1_ref[...], 0.0)
    dec = (jnp.dot(g, wd2_ref[...], preferred_element_type=jnp.float32)
           + bd2_ref[...])
    u = u_ref[...]
    acc = jnp.zeros((bb, DIM), jnp.float32)
    for j in range(DIM):
        acc = acc + u[:, j:j + 1] * dec[:, j * DIM:(j + 1) * DIM]
    out_ref[...] = jnp.sum(acc * iid_ref[...], axis=1)


def _tc_dense(ufea, seq, u, iid, W1, b1, w2, Wd1, bd1, Wd2, bd2, bb=256):
    nblk = BATCH // bb
    full = lambda shape: pl.BlockSpec(shape, lambda i: (0,) * len(shape))
    return pl.pallas_call(
        functools.partial(_tc_body, bb=bb),
        grid=(nblk,),
        in_specs=[
            pl.BlockSpec((bb, HIST, DIM), lambda i: (i, 0, 0)),
            pl.BlockSpec((bb, HIST), lambda i: (i, 0)),
            pl.BlockSpec((bb, DIM), lambda i: (i, 0)),
            pl.BlockSpec((bb, DIM), lambda i: (i, 0)),
            full((DIM, DIM)), full((1, DIM)), full((1, DIM)),
            full((DIM, 2 * DIM)), full((1, 2 * DIM)),
            full((2 * DIM, DIM * DIM)), full((1, DIM * DIM)),
        ],
        out_specs=pl.BlockSpec((bb,), lambda i: (i,)),
        out_shape=jax.ShapeDtypeStruct((BATCH,), jnp.float32),
    )(ufea, seq, u, iid, W1, b1, w2, Wd1, bd1, Wd2, bd2)


def kernel(x, src_uid, src_iid, tgt_iid, W1, b1, W2, Wd1, bd1, Wd2, bd2):
    x = x.astype(jnp.int32)
    seq = x[:, 2:]
    seq_idx = seq.reshape(NW, SEQ_ROWS, 128)
    uidx = x[:, 0].reshape(NW, B_ROWS, 128)
    iidx = x[:, 1].reshape(NW, B_ROWS, 128)

    ufea, u, iid = _sc_gather(seq_idx, uidx, iidx, src_uid, src_iid, tgt_iid)
    ufea = ufea.reshape(BATCH, HIST, DIM)

    return _tc_dense(ufea, seq, u, iid,
                     W1, b1.reshape(1, DIM), W2.reshape(1, DIM),
                     Wd1, bd1.reshape(1, 2 * DIM), Wd2,
                     bd2.reshape(1, DIM * DIM))


# speedup vs baseline: 1.4103x; 1.4103x over previous
import functools

import jax
import jax.numpy as jnp
from jax import lax
from jax.experimental import pallas as pl
from jax.experimental.pallas import tpu as pltpu
from jax.experimental.pallas import tpu_sc as plsc

BATCH = 16384
HIST = 50
DIM = 32
NC = 2
NS = 16
NW = NC * NS

SEQ_TOTAL = BATCH * HIST
SEQ_PER_W = SEQ_TOTAL // NW
SEQ_ROWS = SEQ_PER_W // 128
T_GROUP = 5
INNER = T_GROUP * 4
CHUNK = INNER * 128
OUTER = HIST // T_GROUP
B_PER_W = BATCH // NW
B_ROWS = B_PER_W // 128


def _sc_gather(seq_idx, uidx, iidx, src_uid, src_iid, tgt_iid):
    mesh = plsc.VectorSubcoreMesh(core_axis_name="c", subcore_axis_name="s",
                                  num_cores=NC, num_subcores=NS)

    def body(seq_hbm, uidx_hbm, iidx_hbm, uid_tab, siid_tab, tiid_tab,
             ufea_out, u_out, iid_out,
             idx_v, rows_v, uidx_v, iidx_v, sem):
        wid = lax.axis_index("s") * NC + lax.axis_index("c")

        pltpu.sync_copy(seq_hbm.at[wid], idx_v)
        wb = wid * B_PER_W

        def group(g, carry):
            cps = []
            for j in range(INNER):
                cps.append(pltpu.async_copy(
                    siid_tab.at[idx_v.at[g * INNER + j]],
                    rows_v.at[pl.ds(j * 128, 128)], sem))
            for cp in cps:
                cp.wait()
            for s in range(T_GROUP):
                t = g * T_GROUP + s
                pltpu.sync_copy(
                    rows_v.at[pl.ds(s * B_PER_W, B_PER_W)],
                    ufea_out.at[pl.ds(t * BATCH + wb, B_PER_W)])
            return carry

        lax.fori_loop(0, OUTER, group, 0, unroll=False)

        pltpu.sync_copy(uidx_hbm.at[wid], uidx_v)
        cps = [pltpu.async_copy(uid_tab.at[uidx_v.at[j]],
                                rows_v.at[pl.ds(j * 128, 128)], sem)
               for j in range(B_ROWS)]
        for cp in cps:
            cp.wait()
        pltpu.sync_copy(rows_v.at[pl.ds(0, B_PER_W)],
                        u_out.at[pl.ds(wb, B_PER_W)])

        pltpu.sync_copy(iidx_hbm.at[wid], iidx_v)
        cps = [pltpu.async_copy(tiid_tab.at[iidx_v.at[j]],
                                rows_v.at[pl.ds(j * 128, 128)], sem)
               for j in range(B_ROWS)]
        for cp in cps:
            cp.wait()
        pltpu.sync_copy(rows_v.at[pl.ds(0, B_PER_W)],
                        iid_out.at[pl.ds(wb, B_PER_W)])

    run = pl.kernel(
        body,
        out_type=(
            jax.ShapeDtypeStruct((SEQ_TOTAL, DIM), jnp.float32),
            jax.ShapeDtypeStruct((BATCH, DIM), jnp.float32),
            jax.ShapeDtypeStruct((BATCH, DIM), jnp.float32),
        ),
        mesh=mesh,
        scratch_types=[
            pltpu.VMEM((SEQ_ROWS, 128), jnp.int32),
            pltpu.VMEM((CHUNK, DIM), jnp.float32),
            pltpu.VMEM((B_ROWS, 128), jnp.int32),
            pltpu.VMEM((B_ROWS, 128), jnp.int32),
            pltpu.SemaphoreType.DMA,
        ],
        compiler_params=pltpu.CompilerParams(use_tc_tiling_on_sc=False),
    )
    return run(seq_idx, uidx, iidx, src_uid, src_iid, tgt_iid)


def _tc_body(ufea_ref, seq_ref, u_ref, iid_ref,
             w1_ref, b1_ref, w2_ref, wd1_ref, bd1_ref, wd2_ref, bd2_ref,
             out_ref, *, bb):
    uf3 = ufea_ref[...]
    uf = uf3.reshape(HIST * bb, DIM)
    h = jnp.maximum(
        jnp.dot(uf, w1_ref[...], preferred_element_type=jnp.float32)
        + b1_ref[...], 0.0)
    e = jnp.sum(h * w2_ref[...], axis=1).reshape(HIST, bb)
    t = e - (seq_ref[...] == 0).astype(jnp.float32) * 1e8
    m = jnp.max(t, axis=0, keepdims=True)
    p = jnp.exp(t - m)
    att = p / jnp.sum(p, axis=0, keepdims=True)
    his = jnp.sum(uf3 * att[:, :, None], axis=0)
    g = jnp.maximum(
        jnp.dot(his, wd1_ref[...], preferred_element_type=jnp.float32)
        + bd1_ref[...], 0.0)
    dec = (jnp.dot(g, wd2_ref[...], preferred_element_type=jnp.float32)
           + bd2_ref[...])
    u = u_ref[...]
    acc = jnp.zeros((bb, DIM), jnp.float32)
    for j in range(DIM):
        acc = acc + u[:, j:j + 1] * dec[:, j * DIM:(j + 1) * DIM]
    out_ref[...] = jnp.sum(acc * iid_ref[...], axis=1)


def _tc_dense(ufea, seqT, u, iid, W1, b1, w2, Wd1, bd1, Wd2, bd2, bb=256):
    nblk = BATCH // bb
    full = lambda shape: pl.BlockSpec(shape, lambda i: (0,) * len(shape))
    return pl.pallas_call(
        functools.partial(_tc_body, bb=bb),
        grid=(nblk,),
        in_specs=[
            pl.BlockSpec((HIST, bb, DIM), lambda i: (0, i, 0)),
            pl.BlockSpec((HIST, bb), lambda i: (0, i)),
            pl.BlockSpec((bb, DIM), lambda i: (i, 0)),
            pl.BlockSpec((bb, DIM), lambda i: (i, 0)),
            full((DIM, DIM)), full((1, DIM)), full((1, DIM)),
            full((DIM, 2 * DIM)), full((1, 2 * DIM)),
            full((2 * DIM, DIM * DIM)), full((1, DIM * DIM)),
        ],
        out_specs=pl.BlockSpec((bb,), lambda i: (i,)),
        out_shape=jax.ShapeDtypeStruct((BATCH,), jnp.float32),
    )(ufea, seqT, u, iid, W1, b1, w2, Wd1, bd1, Wd2, bd2)


def kernel(x, src_uid, src_iid, tgt_iid, W1, b1, W2, Wd1, bd1, Wd2, bd2):
    x = x.astype(jnp.int32)
    seqT = x[:, 2:].T
    seq_idx = seqT.reshape(HIST, NW, B_PER_W).transpose(1, 0, 2) \
                  .reshape(NW, SEQ_ROWS, 128)
    uidx = x[:, 0].reshape(NW, B_ROWS, 128)
    iidx = x[:, 1].reshape(NW, B_ROWS, 128)

    ufea, u, iid = _sc_gather(seq_idx, uidx, iidx, src_uid, src_iid, tgt_iid)
    ufea = ufea.reshape(HIST, BATCH, DIM)

    return _tc_dense(ufea, seqT, u, iid,
                     W1, b1.reshape(1, DIM), W2.reshape(1, DIM),
                     Wd1, bd1.reshape(1, 2 * DIM), Wd2,
                     bd2.reshape(1, DIM * DIM))
